# baseline (speedup 1.0000x reference)
import functools

import jax
import jax.numpy as jnp
from jax import lax
from jax.experimental import pallas as pl
from jax.experimental.pallas import tpu as pltpu
from jax.experimental.pallas import tpu_sc as plsc

_TABLE = 1000000
_D = 32
_BATCH = 16384
_HIST = 50

_NC, _NS = 2, 16
_NW = _NC * _NS
_B_TOTAL = _BATCH * _HIST
_CHUNK = 128
_N_CHUNKS = _B_TOTAL // (_NW * _CHUNK)
_NBUF = 2


def _reduce_chunk(rows_v, buf, out_v, j):
    inv = jnp.float32(1.0 / _D)
    buf_ids = jnp.broadcast_to(buf, (16,)).astype(jnp.int32)
    for g in range(_CHUNK // 16):
        row_ids = lax.iota(jnp.int32, 16) + (16 * g)
        acc = plsc.load_gather(
            rows_v, [buf_ids, row_ids, jnp.zeros((16,), jnp.int32)])
        for d in range(1, _D):
            acc = acc + plsc.load_gather(
                rows_v, [buf_ids, row_ids, jnp.full((16,), d, jnp.int32)])
        out_v[j, pl.ds(16 * g, 16)] = acc * inv


def _gather_body(tbl_hbm, idx_hbm, out_hbm, idx_v, rows_v, out_v, sem):
    wid = lax.axis_index("s") * _NC + lax.axis_index("c")
    pltpu.sync_copy(idx_hbm.at[wid], idx_v)

    pltpu.async_copy(tbl_hbm.at[idx_v.at[0]], rows_v.at[0], sem)

    def step(j, _):
        cur = lax.rem(j, _NBUF)
        nxt = lax.rem(j + 1, _NBUF)

        @pl.when(j + 1 < _N_CHUNKS)
        def _prefetch():
            pltpu.async_copy(tbl_hbm.at[idx_v.at[j + 1]], rows_v.at[nxt], sem)

        pltpu.make_async_copy(
            tbl_hbm.at[idx_v.at[j]], rows_v.at[cur], sem).wait()
        _reduce_chunk(rows_v, cur, out_v, j)
        return _

    lax.fori_loop(0, _N_CHUNKS, step, None)
    pltpu.sync_copy(out_v, out_hbm.at[wid])


def _sc_rowmean_gather(embedding, idx3):
    mesh = plsc.VectorSubcoreMesh(core_axis_name="c", subcore_axis_name="s")
    f = pl.kernel(
        _gather_body,
        out_type=jax.ShapeDtypeStruct((_NW, _N_CHUNKS, _CHUNK), jnp.float32),
        mesh=mesh,
        compiler_params=pltpu.CompilerParams(
            needs_layout_passes=False, use_tc_tiling_on_sc=False),
        scratch_types=[
            pltpu.VMEM((_N_CHUNKS, _CHUNK), jnp.int32),
            pltpu.VMEM((_NBUF, _CHUNK, _D), jnp.float32),
            pltpu.VMEM((_N_CHUNKS, _CHUNK), jnp.float32),
            pltpu.SemaphoreType.DMA,
        ],
    )
    return f(embedding, idx3)


def kernel(inputs, embedding):
    idx3 = inputs.reshape(_NW, _N_CHUNKS, _CHUNK)
    out = _sc_rowmean_gather(embedding, idx3)
    return out.reshape(_BATCH, _HIST)


# speedup vs baseline: 4.4757x; 4.4757x over previous
import functools

import jax
import jax.numpy as jnp
from jax import lax
from jax.experimental import pallas as pl
from jax.experimental.pallas import tpu as pltpu
from jax.experimental.pallas import tpu_sc as plsc

_TABLE = 1000000
_D = 32
_BATCH = 16384
_HIST = 50


_BL = 8192


def _mean_body(x_ref, o_ref):
    o_ref[...] = jnp.sum(x_ref[...], axis=0) * (1.0 / _D)


def _row_means(emb_t):
    return pl.pallas_call(
        _mean_body,
        grid=((_TABLE + _BL - 1) // _BL,),
        in_specs=[pl.BlockSpec((_D, _BL), lambda i: (0, i))],
        out_specs=pl.BlockSpec((_BL,), lambda i: (i,)),
        out_shape=jax.ShapeDtypeStruct((_TABLE,), jnp.float32),
    )(emb_t)


_NC, _NS = 2, 16
_NW = _NC * _NS
_B_TOTAL = _BATCH * _HIST
_CHUNK = 128
_N_CHUNKS = _B_TOTAL // (_NW * _CHUNK)
_FIRE = 8


def _gather_body(means_hbm, idx_hbm, out_hbm, idx_v, vals_v, sem):
    wid = lax.axis_index("s") * _NC + lax.axis_index("c")
    pltpu.sync_copy(idx_hbm.at[wid], idx_v)

    def outer(o, _):
        for b in range(_FIRE):
            j = o * _FIRE + b
            pltpu.async_copy(means_hbm.at[idx_v.at[j]], vals_v.at[j], sem)
        for b in range(_FIRE):
            j = o * _FIRE + b
            pltpu.make_async_copy(
                means_hbm.at[idx_v.at[j]], vals_v.at[j], sem).wait()
        return _

    lax.fori_loop(0, _N_CHUNKS // _FIRE, outer, None)
    pltpu.sync_copy(vals_v, out_hbm.at[wid])


def _sc_gather(means, idx3):
    mesh = plsc.VectorSubcoreMesh(core_axis_name="c", subcore_axis_name="s")
    f = pl.kernel(
        _gather_body,
        out_type=jax.ShapeDtypeStruct((_NW, _N_CHUNKS, _CHUNK), jnp.float32),
        mesh=mesh,
        scratch_types=[
            pltpu.VMEM((_N_CHUNKS, _CHUNK), jnp.int32),
            pltpu.VMEM((_N_CHUNKS, _CHUNK), jnp.float32),
            pltpu.SemaphoreType.DMA,
        ],
    )
    return f(means, idx3)


def kernel(inputs, embedding):
    means = _row_means(embedding.T)
    idx3 = inputs.reshape(_NW, _N_CHUNKS, _CHUNK)
    out = _sc_gather(means, idx3)
    return out.reshape(_BATCH, _HIST)


# speedup vs baseline: 6.2340x; 1.3929x over previous
import functools

import jax
import jax.numpy as jnp
from jax import lax
from jax.experimental import pallas as pl
from jax.experimental.pallas import tpu as pltpu
from jax.experimental.pallas import tpu_sc as plsc

_TABLE = 1000000
_D = 32
_BATCH = 16384
_HIST = 50


_BL = 32768


def _mean_body(x_ref, o_ref):
    o_ref[...] = jnp.sum(x_ref[...], axis=0) * (1.0 / _D)


def _row_means(emb_t):
    return pl.pallas_call(
        _mean_body,
        grid=((_TABLE + _BL - 1) // _BL,),
        in_specs=[pl.BlockSpec((_D, _BL), lambda i: (0, i))],
        out_specs=pl.BlockSpec((_BL,), lambda i: (i,)),
        out_shape=jax.ShapeDtypeStruct((_TABLE,), jnp.float32),
    )(emb_t)


_NC, _NS = 2, 16
_NW = _NC * _NS
_B_TOTAL = _BATCH * _HIST
_CHUNK = 128
_N_CHUNKS = _B_TOTAL // (_NW * _CHUNK)
_FIRE = 20


def _gather_body(means_hbm, idx_hbm, out_hbm, idx_v, vals_v, sem):
    wid = lax.axis_index("s") * _NC + lax.axis_index("c")
    pltpu.sync_copy(idx_hbm.at[wid], idx_v)

    def outer(o, _):
        for b in range(_FIRE):
            j = o * _FIRE + b
            pltpu.async_copy(means_hbm.at[idx_v.at[j]], vals_v.at[j], sem)
        for b in range(_FIRE):
            j = o * _FIRE + b
            pltpu.make_async_copy(
                means_hbm.at[idx_v.at[j]], vals_v.at[j], sem).wait()
        return _

    lax.fori_loop(0, _N_CHUNKS // _FIRE, outer, None)
    pltpu.sync_copy(vals_v, out_hbm.at[wid])


def _sc_gather(means, idx3):
    mesh = plsc.VectorSubcoreMesh(core_axis_name="c", subcore_axis_name="s")
    f = pl.kernel(
        _gather_body,
        out_type=jax.ShapeDtypeStruct((_NW, _N_CHUNKS, _CHUNK), jnp.float32),
        mesh=mesh,
        scratch_types=[
            pltpu.VMEM((_N_CHUNKS, _CHUNK), jnp.int32),
            pltpu.VMEM((_N_CHUNKS, _CHUNK), jnp.float32),
            pltpu.SemaphoreType.DMA,
        ],
    )
    return f(means, idx3)


def kernel(inputs, embedding):
    means = _row_means(embedding.T)
    idx3 = inputs.reshape(_NW, _N_CHUNKS, _CHUNK)
    out = _sc_gather(means, idx3)
    return out.reshape(_BATCH, _HIST)
